# baseline (speedup 1.0000x reference)
import math

import jax
import jax.numpy as jnp
from jax.experimental import pallas as pl
from jax.experimental.pallas import tpu as pltpu

S = 2048
H = 768
NH = 12
NKV = 4
HD = 64
HALF = HD // 2
I = 768
E = 64
THETA = 150000.0
EPS = 1e-6
BLK = 256
GRID = S // BLK
REP = NH // NKV
R = REP * BLK
CH = 512
NCH = S // CH
Q_SIZE = NH * HD
KV_SIZE = NKV * HD

_NEG = -1e30
_NT = (((1,), (1,)), ((), ()))


def _body(pos_ref, hid_ref, res_ref, wqkv_ref, bqkv_ref, ln1_ref,
          wo_ref, bo_ref, ln2_ref, wr_ref, br_ref,
          wgu_ref, bgu_ref, wd_ref, bd_ref,
          out_ref, r2_out,
          qst, kst, vs, r1s, mbuf, lbuf, obuf,
          wqkv_bf, wo_bf, wr_bf, wgu_bf, wd_bf):
    j = pl.program_id(0)
    t = pl.program_id(1)

    @pl.when(jnp.logical_and(j == 0, t == 0))
    def _init_once():
        wqkv_bf[...] = wqkv_ref[...].astype(jnp.bfloat16)
        wo_bf[...] = wo_ref[...].astype(jnp.bfloat16)
        wr_bf[...] = wr_ref[...].astype(jnp.bfloat16)
        wgu_bf[...] = wgu_ref[...].astype(jnp.bfloat16)
        wd_bf[...] = wd_ref[...].astype(jnp.bfloat16)
        vs[...] = jnp.zeros((S, KV_SIZE), jnp.bfloat16)
        obuf[...] = jnp.zeros((R, NKV * NCH * HD), jnp.float32)

    @pl.when(t == 0)
    def _phase_a():
        x = hid_ref[...] + res_ref[...]
        r1s[pl.ds(j * BLK, BLK), :] = x
        ms = jnp.mean(x * x, axis=1, keepdims=True)
        h = x * jax.lax.rsqrt(ms + EPS) * ln1_ref[...]
        qkv = jax.lax.dot_general(
            h.astype(jnp.bfloat16), wqkv_bf[...], _NT,
            preferred_element_type=jnp.float32) + bqkv_ref[...]

        pos = pos_ref[...]
        jq = jax.lax.rem(
            jax.lax.broadcasted_iota(jnp.int32, (1, NH * HALF), 1),
            HALF).astype(jnp.float32)
        inv_freq = jnp.exp(jq * (-math.log(THETA) / HALF))
        f = pos * inv_freq
        cos_q = jnp.cos(f)
        sin_q = jnp.sin(f)
        cos_k = cos_q[:, :NKV * HALF]
        sin_k = sin_q[:, :NKV * HALF]

        def split(x2, nheads):
            h1 = [x2[:, hh * HD:hh * HD + HALF] for hh in range(nheads)]
            h2 = [x2[:, hh * HD + HALF:(hh + 1) * HD] for hh in range(nheads)]
            return (jnp.concatenate(h1, axis=1), jnp.concatenate(h2, axis=1))

        q1, q2 = split(qkv[:, :Q_SIZE], NH)
        k1, k2 = split(qkv[:, Q_SIZE:Q_SIZE + KV_SIZE], NKV)

        scale = HD ** -0.5 * math.log2(math.e)
        q1r = (q1 * cos_q - q2 * sin_q) * scale
        q2r = (q2 * cos_q + q1 * sin_q) * scale
        k1r = k1 * cos_k - k2 * sin_k
        k2r = k2 * cos_k + k1 * sin_k

        rows = []
        for hh in range(REP):
            rows.append(jnp.concatenate(
                [jnp.concatenate(
                    [q1r[:, hq * HALF:(hq + 1) * HALF],
                     q2r[:, hq * HALF:(hq + 1) * HALF]], axis=1)
                 for hq in range(hh, NH, REP)], axis=1))
        qst[pl.ds(j * R, R), :] = jnp.concatenate(
            rows, axis=0).astype(jnp.bfloat16)
        kst[pl.ds(j * BLK, BLK), :] = jnp.concatenate(
            [jnp.concatenate(
                [k1r[:, g * HALF:(g + 1) * HALF],
                 k2r[:, g * HALF:(g + 1) * HALF]], axis=1)
             for g in range(NKV)], axis=1).astype(jnp.bfloat16)
        vs[pl.ds(j * BLK, BLK), :] = qkv[:, Q_SIZE + KV_SIZE:].astype(
            jnp.bfloat16)

        mbuf[...] = jnp.full((R, NKV * NCH), _NEG, jnp.float32)
        lbuf[...] = jnp.zeros((R, NKV * NCH), jnp.float32)

    def do_chunk(c, masked):
        for g in range(NKV):
            q_g = qst[pl.ds(j * R, R), g * HD:(g + 1) * HD]
            k_c = kst[pl.ds(c * CH, CH), g * HD:(g + 1) * HD]
            v_c = vs[pl.ds(c * CH, CH), g * HD:(g + 1) * HD]
            s = jax.lax.dot_general(q_g, k_c, _NT,
                                    preferred_element_type=jnp.float32)
            if masked:
                row = jax.lax.rem(
                    jax.lax.broadcasted_iota(jnp.int32, (R, 1), 0), BLK)
                col = jax.lax.broadcasted_iota(jnp.int32, (1, CH), 1)
                s = jnp.where((c * CH + col) <= (j * BLK + row), s, _NEG)
            m_c = jnp.max(s, axis=1, keepdims=True)
            p = jnp.exp2(s - m_c)
            l_c = jnp.sum(p, axis=1, keepdims=True)
            o_c = jnp.dot(p.astype(jnp.bfloat16), v_c,
                          preferred_element_type=jnp.float32)
            slot = g * NCH + c
            mbuf[:, slot:slot + 1] = m_c
            lbuf[:, slot:slot + 1] = l_c
            obuf[:, slot * HD:(slot + 1) * HD] = o_c

    def finalize_and_epilogue():
        o_cols = [None] * NH
        for g in range(NKV):
            mrow = mbuf[:, g * NCH:(g + 1) * NCH]
            m = jnp.max(mrow, axis=1, keepdims=True)
            w = jnp.exp2(mrow - m)
            l = jnp.sum(w * lbuf[:, g * NCH:(g + 1) * NCH],
                        axis=1, keepdims=True)
            o_acc = jnp.zeros((R, HD), jnp.float32)
            for c in range(NCH):
                slot = g * NCH + c
                o_acc = o_acc + w[:, c:c + 1] * obuf[:, slot * HD:
                                                     (slot + 1) * HD]
            o_g = o_acc / l
            for hh in range(REP):
                o_cols[g * REP + hh] = o_g[hh * BLK:(hh + 1) * BLK, :]
        o = jnp.concatenate(o_cols, axis=1).astype(jnp.bfloat16)

        attn = jax.lax.dot_general(
            o, wo_bf[...], _NT,
            preferred_element_type=jnp.float32) + bo_ref[...]
        r2 = attn + r1s[pl.ds(j * BLK, BLK), :]
        r2_out[...] = r2

        ms = jnp.mean(r2 * r2, axis=1, keepdims=True)
        h2 = (r2 * jax.lax.rsqrt(ms + EPS) * ln2_ref[...]).astype(
            jnp.bfloat16)

        logits = jax.lax.dot_general(
            h2, wr_bf[...], _NT,
            preferred_element_type=jnp.float32) + br_ref[...]
        m1 = jnp.max(logits, axis=1, keepdims=True)
        s2 = jnp.max(jnp.where(logits >= m1, _NEG, logits),
                     axis=1, keepdims=True)
        e2 = jnp.exp(s2 - m1)
        denom = 1.0 + e2
        factor = 1.0 / denom + e2 / denom

        gu = jax.lax.dot_general(
            h2, wgu_bf[...], _NT,
            preferred_element_type=jnp.float32) + bgu_ref[...]
        gate = gu[:, :I]
        up = gu[:, I:]
        x = gate * (up * jax.nn.sigmoid(up))
        eo = jax.lax.dot_general(
            x.astype(jnp.bfloat16), wd_bf[...], _NT,
            preferred_element_type=jnp.float32) + bd_ref[...]
        out_ref[...] = factor * eo

    for c in range(NCH):
        @pl.when(jnp.logical_and(t == c + 1, c + 1 <= jnp.floor_divide(j, 2)))
        def _off_diag(c=c):
            do_chunk(c, masked=False)

        @pl.when(jnp.logical_and(t == c + 1, c == jnp.floor_divide(j, 2)))
        def _diag(c=c):
            do_chunk(c, masked=True)
            finalize_and_epilogue()


def kernel(positions, hidden_states, residual, w_qkv, b_qkv, w_o, b_o,
           w_router, b_router, w_gate_up, b_gate_up, w_down, b_down,
           ln1_w, ln2_w):
    f32 = jnp.float32
    bf16 = jnp.bfloat16
    pos = positions.astype(f32).reshape(S, 1)

    full = lambda shape: pl.BlockSpec(shape, lambda j, t: (0, 0))
    blk_j = lambda cols: pl.BlockSpec((BLK, cols), lambda j, t: (j, 0))

    out, r2 = pl.pallas_call(
        _body,
        grid=(GRID, NCH + 1),
        in_specs=[
            blk_j(1),
            blk_j(H),
            blk_j(H),
            full((Q_SIZE + 2 * KV_SIZE, H)),
            full((1, Q_SIZE + 2 * KV_SIZE)),
            full((1, H)),
            full((H, Q_SIZE)),
            full((1, H)),
            full((1, H)),
            full((E, H)),
            full((1, E)),
            full((2 * I, H)),
            full((1, 2 * I)),
            full((H, I)),
            full((1, H)),
        ],
        out_specs=[blk_j(H), blk_j(H)],
        out_shape=[
            jax.ShapeDtypeStruct((S, H), f32),
            jax.ShapeDtypeStruct((S, H), f32),
        ],
        scratch_shapes=[
            pltpu.VMEM((GRID * R, KV_SIZE), bf16),
            pltpu.VMEM((S, KV_SIZE), bf16),
            pltpu.VMEM((S, KV_SIZE), bf16),
            pltpu.VMEM((S, H), f32),
            pltpu.VMEM((R, NKV * NCH), f32),
            pltpu.VMEM((R, NKV * NCH), f32),
            pltpu.VMEM((R, NKV * NCH * HD), f32),
            pltpu.VMEM((Q_SIZE + 2 * KV_SIZE, H), bf16),
            pltpu.VMEM((H, Q_SIZE), bf16),
            pltpu.VMEM((E, H), bf16),
            pltpu.VMEM((2 * I, H), bf16),
            pltpu.VMEM((H, I), bf16),
        ],
    )(pos, hidden_states, residual, w_qkv,
      b_qkv.reshape(1, -1).astype(f32), ln1_w.reshape(1, H).astype(f32),
      w_o, b_o.reshape(1, H).astype(f32), ln2_w.reshape(1, H).astype(f32),
      w_router, b_router.reshape(1, E).astype(f32),
      w_gate_up, b_gate_up.reshape(1, 2 * I).astype(f32),
      w_down, b_down.reshape(1, H).astype(f32))

    return (out, r2)


# speedup vs baseline: 12.3036x; 12.3036x over previous
import math

import jax
import jax.numpy as jnp
from jax.experimental import pallas as pl
from jax.experimental.pallas import tpu as pltpu

S = 2048
H = 768
NH = 12
NKV = 4
HD = 64
HALF = HD // 2
I = 768
E = 64
THETA = 150000.0
EPS = 1e-6
BLK = 256
GRID = S // BLK
REP = NH // NKV
Q_SIZE = NH * HD
KV_SIZE = NKV * HD

_NEG = -1e30
_NT = (((1,), (1,)), ((), ()))


def _split_halves(x, nheads):
    h1 = [x[:, h * HD:h * HD + HALF] for h in range(nheads)]
    h2 = [x[:, h * HD + HALF:(h + 1) * HD] for h in range(nheads)]
    return jnp.concatenate(h1 + h2, axis=1)


def _body(pos_ref, hid_ref, res_ref, wqkv_ref, bqkv_ref, ln1_ref,
          wo_ref, bo_ref, ln2_ref, wr_ref, br_ref,
          wgu_ref, bgu_ref, wd_ref, bd_ref,
          out_ref, r2_out,
          qs, ks, vs, r1s, wqkv_bf, wo_bf, wr_bf, wgu_bf, wd_bf):
    i = pl.program_id(0)

    @pl.when(i == 0)
    def _cast_weights():
        wqkv_bf[...] = wqkv_ref[...].astype(jnp.bfloat16)
        wo_bf[...] = wo_ref[...].astype(jnp.bfloat16)
        wr_bf[...] = wr_ref[...].astype(jnp.bfloat16)
        wgu_bf[...] = wgu_ref[...].astype(jnp.bfloat16)
        wd_bf[...] = wd_ref[...].astype(jnp.bfloat16)

    @pl.when(i < GRID)
    def _phase_a():
        x = hid_ref[...] + res_ref[...]
        rows = pl.ds(i * BLK, BLK)
        r1s[rows, :] = x
        ms = jnp.mean(x * x, axis=1, keepdims=True)
        h = x * jax.lax.rsqrt(ms + EPS) * ln1_ref[...]
        qkv = jax.lax.dot_general(
            h.astype(jnp.bfloat16), wqkv_bf[...], _NT,
            preferred_element_type=jnp.float32) + bqkv_ref[...]

        pos = pos_ref[...]
        jq = jax.lax.rem(
            jax.lax.broadcasted_iota(jnp.int32, (1, NH * HALF), 1),
            HALF).astype(jnp.float32)
        inv_freq = jnp.exp(jq * (-math.log(THETA) / HALF))
        f = pos * inv_freq
        cos_q = jnp.cos(f)
        sin_q = jnp.sin(f)
        cos_k = cos_q[:, :NKV * HALF]
        sin_k = sin_q[:, :NKV * HALF]

        qh = _split_halves(qkv[:, :Q_SIZE], NH)
        kh = _split_halves(qkv[:, Q_SIZE:Q_SIZE + KV_SIZE], NKV)
        q1 = qh[:, :NH * HALF]
        q2 = qh[:, NH * HALF:]
        k1 = kh[:, :NKV * HALF]
        k2 = kh[:, NKV * HALF:]

        scale = HD ** -0.5 * math.log2(math.e)
        qs[rows, :] = (jnp.concatenate(
            [q1 * cos_q - q2 * sin_q, q2 * cos_q + q1 * sin_q],
            axis=1) * scale).astype(jnp.bfloat16)
        ks[rows, :] = jnp.concatenate(
            [k1 * cos_k - k2 * sin_k, k2 * cos_k + k1 * sin_k],
            axis=1).astype(jnp.bfloat16)
        vs[rows, :] = qkv[:, Q_SIZE + KV_SIZE:].astype(jnp.bfloat16)

    @pl.when(i >= GRID)
    def _phase_b():
        j = i - GRID
        q0 = j * BLK
        R = REP * BLK
        qrows = pl.ds(q0, BLK)

        row = jax.lax.rem(
            jax.lax.broadcasted_iota(jnp.int32, (R, 1), 0), BLK)
        col = jax.lax.broadcasted_iota(jnp.int32, (1, S), 1)
        mask = col <= (q0 + row)

        q_blk = qs[qrows, :]
        o_cols = []
        for g in range(NKV):
            hs = []
            for hh in range(REP):
                h = g * REP + hh
                hs.append(jnp.concatenate(
                    [q_blk[:, h * HALF:(h + 1) * HALF],
                     q_blk[:, NH * HALF + h * HALF:
                           NH * HALF + (h + 1) * HALF]],
                    axis=1))
            q_g = jnp.concatenate(hs, axis=0)

            k_g = jnp.concatenate(
                [ks[:, g * HALF:(g + 1) * HALF],
                 ks[:, NKV * HALF + g * HALF:NKV * HALF + (g + 1) * HALF]],
                axis=1)
            v_g = vs[:, g * HD:(g + 1) * HD]
            s = jax.lax.dot_general(q_g, k_g, _NT,
                                    preferred_element_type=jnp.float32)
            s = jnp.where(mask, s, _NEG)
            m = jnp.max(s, axis=1, keepdims=True)
            p = jnp.exp2(s - m)
            l = jnp.sum(p, axis=1, keepdims=True)
            o_g = jnp.dot(p.astype(jnp.bfloat16), v_g,
                          preferred_element_type=jnp.float32) / l
            for hh in range(REP):
                o_cols.append(o_g[hh * BLK:(hh + 1) * BLK, :])
        o = jnp.concatenate(o_cols, axis=1).astype(jnp.bfloat16)

        attn = jax.lax.dot_general(
            o, wo_bf[...], _NT,
            preferred_element_type=jnp.float32) + bo_ref[...]
        r2 = attn + r1s[qrows, :]
        r2_out[...] = r2

        ms = jnp.mean(r2 * r2, axis=1, keepdims=True)
        h2 = (r2 * jax.lax.rsqrt(ms + EPS) * ln2_ref[...]).astype(
            jnp.bfloat16)

        logits = jax.lax.dot_general(
            h2, wr_bf[...], _NT,
            preferred_element_type=jnp.float32) + br_ref[...]
        m1 = jnp.max(logits, axis=1, keepdims=True)
        s2 = jnp.max(jnp.where(logits >= m1, _NEG, logits),
                     axis=1, keepdims=True)
        e2 = jnp.exp(s2 - m1)
        denom = 1.0 + e2
        factor = 1.0 / denom + e2 / denom

        gu = jax.lax.dot_general(
            h2, wgu_bf[...], _NT,
            preferred_element_type=jnp.float32) + bgu_ref[...]
        gate = gu[:, :I]
        up = gu[:, I:]
        x = gate * (up * jax.nn.sigmoid(up))
        eo = jax.lax.dot_general(
            x.astype(jnp.bfloat16), wd_bf[...], _NT,
            preferred_element_type=jnp.float32) + bd_ref[...]
        out_ref[...] = factor * eo


def kernel(positions, hidden_states, residual, w_qkv, b_qkv, w_o, b_o,
           w_router, b_router, w_gate_up, b_gate_up, w_down, b_down,
           ln1_w, ln2_w):
    f32 = jnp.float32
    bf16 = jnp.bfloat16
    pos = positions.astype(f32).reshape(S, 1)

    full = lambda shape: pl.BlockSpec(shape, lambda i: (0, 0))
    blk_a = lambda cols: pl.BlockSpec(
        (BLK, cols), lambda i: (jnp.minimum(i, GRID - 1), 0))
    blk_b = lambda cols: pl.BlockSpec(
        (BLK, cols), lambda i: (jnp.maximum(i - GRID, 0), 0))

    out, r2 = pl.pallas_call(
        _body,
        grid=(2 * GRID,),
        in_specs=[
            blk_a(1),
            blk_a(H),
            blk_a(H),
            full((Q_SIZE + 2 * KV_SIZE, H)),
            full((1, Q_SIZE + 2 * KV_SIZE)),
            full((1, H)),
            full((H, Q_SIZE)),
            full((1, H)),
            full((1, H)),
            full((E, H)),
            full((1, E)),
            full((2 * I, H)),
            full((1, 2 * I)),
            full((H, I)),
            full((1, H)),
        ],
        out_specs=[blk_b(H), blk_b(H)],
        out_shape=[
            jax.ShapeDtypeStruct((S, H), f32),
            jax.ShapeDtypeStruct((S, H), f32),
        ],
        scratch_shapes=[
            pltpu.VMEM((S, Q_SIZE), bf16),
            pltpu.VMEM((S, KV_SIZE), bf16),
            pltpu.VMEM((S, KV_SIZE), bf16),
            pltpu.VMEM((S, H), f32),
            pltpu.VMEM((Q_SIZE + 2 * KV_SIZE, H), bf16),
            pltpu.VMEM((H, Q_SIZE), bf16),
            pltpu.VMEM((E, H), bf16),
            pltpu.VMEM((2 * I, H), bf16),
            pltpu.VMEM((H, I), bf16),
        ],
    )(pos, hidden_states, residual, w_qkv,
      b_qkv.reshape(1, -1).astype(f32), ln1_w.reshape(1, H).astype(f32),
      w_o, b_o.reshape(1, H).astype(f32), ln2_w.reshape(1, H).astype(f32),
      w_router, b_router.reshape(1, E).astype(f32),
      w_gate_up, b_gate_up.reshape(1, 2 * I).astype(f32),
      w_down, b_down.reshape(1, H).astype(f32))

    return (out, r2)


# speedup vs baseline: 14.4783x; 1.1768x over previous
import math

import jax
import jax.numpy as jnp
from jax.experimental import pallas as pl
from jax.experimental.pallas import tpu as pltpu

S = 2048
H = 768
NH = 12
NKV = 4
HD = 64
HALF = HD // 2
I = 768
E = 64
THETA = 150000.0
EPS = 1e-6
BLK = 256
GRID = S // BLK
REP = NH // NKV
Q_SIZE = NH * HD
KV_SIZE = NKV * HD

_NEG = -1e30
_NT = (((1,), (1,)), ((), ()))


def _split_halves(x, nheads):
    h1 = [x[:, h * HD:h * HD + HALF] for h in range(nheads)]
    h2 = [x[:, h * HD + HALF:(h + 1) * HD] for h in range(nheads)]
    return jnp.concatenate(h1 + h2, axis=1)


def _body(pos_ref, hid_ref, res_ref, wqkv_ref, bqkv_ref, ln1_ref,
          wo_ref, bo_ref, ln2_ref, wr_ref, br_ref,
          wgu_ref, bgu_ref, wd_ref, bd_ref,
          out_ref, r2_out,
          qs, ks, vs, r1s, o_sc, wqkv_bf, wo_bf, wr_bf, wgu_bf, wd_bf):
    i = pl.program_id(0)

    @pl.when(i == 0)
    def _cast_weights():
        wqkv_bf[...] = wqkv_ref[...].astype(jnp.bfloat16)
        wo_bf[...] = wo_ref[...].astype(jnp.bfloat16)
        wr_bf[...] = wr_ref[...].astype(jnp.bfloat16)
        wgu_bf[...] = wgu_ref[...].astype(jnp.bfloat16)
        wd_bf[...] = wd_ref[...].astype(jnp.bfloat16)

    @pl.when(i < GRID)
    def _phase_a():
        x = hid_ref[...] + res_ref[...]
        rows = pl.ds(i * BLK, BLK)
        r1s[rows, :] = x
        ms = jnp.mean(x * x, axis=1, keepdims=True)
        h = x * jax.lax.rsqrt(ms + EPS) * ln1_ref[...]
        qkv = jax.lax.dot_general(
            h.astype(jnp.bfloat16), wqkv_bf[...], _NT,
            preferred_element_type=jnp.float32) + bqkv_ref[...]

        pos = pos_ref[...]
        jq = jax.lax.rem(
            jax.lax.broadcasted_iota(jnp.int32, (1, NH * HALF), 1),
            HALF).astype(jnp.float32)
        inv_freq = jnp.exp(jq * (-math.log(THETA) / HALF))
        f = pos * inv_freq
        cos_q = jnp.cos(f)
        sin_q = jnp.sin(f)
        cos_k = cos_q[:, :NKV * HALF]
        sin_k = sin_q[:, :NKV * HALF]

        qh = _split_halves(qkv[:, :Q_SIZE], NH)
        kh = _split_halves(qkv[:, Q_SIZE:Q_SIZE + KV_SIZE], NKV)
        q1 = qh[:, :NH * HALF]
        q2 = qh[:, NH * HALF:]
        k1 = kh[:, :NKV * HALF]
        k2 = kh[:, NKV * HALF:]

        scale = HD ** -0.5 * math.log2(math.e)
        qs[rows, :] = (jnp.concatenate(
            [q1 * cos_q - q2 * sin_q, q2 * cos_q + q1 * sin_q],
            axis=1) * scale).astype(jnp.bfloat16)
        ks[rows, :] = jnp.concatenate(
            [k1 * cos_k - k2 * sin_k, k2 * cos_k + k1 * sin_k],
            axis=1).astype(jnp.bfloat16)
        vs[rows, :] = qkv[:, Q_SIZE + KV_SIZE:].astype(jnp.bfloat16)

    def _attn(W):
        j = i - GRID
        q0 = j * BLK
        R = REP * BLK

        row = jax.lax.rem(
            jax.lax.broadcasted_iota(jnp.int32, (R, 1), 0), BLK)
        col = jax.lax.broadcasted_iota(jnp.int32, (1, W), 1)
        mask = col <= (q0 + row)

        q_blk = qs[pl.ds(q0, BLK), :]
        o_cols = []
        for g in range(NKV):
            hs = []
            for hh in range(REP):
                h = g * REP + hh
                hs.append(jnp.concatenate(
                    [q_blk[:, h * HALF:(h + 1) * HALF],
                     q_blk[:, NH * HALF + h * HALF:
                           NH * HALF + (h + 1) * HALF]],
                    axis=1))
            q_g = jnp.concatenate(hs, axis=0)

            k_g = jnp.concatenate(
                [ks[:W, g * HALF:(g + 1) * HALF],
                 ks[:W, NKV * HALF + g * HALF:NKV * HALF + (g + 1) * HALF]],
                axis=1)
            v_g = vs[:W, g * HD:(g + 1) * HD]
            s = jax.lax.dot_general(q_g, k_g, _NT,
                                    preferred_element_type=jnp.float32)
            s = jnp.where(mask, s, _NEG)
            m = jnp.max(s, axis=1, keepdims=True)
            p = jnp.exp2(s - m)
            l = jnp.sum(p, axis=1, keepdims=True)
            o_g = jnp.dot(p.astype(jnp.bfloat16), v_g,
                          preferred_element_type=jnp.float32) / l
            for hh in range(REP):
                o_cols.append(o_g[hh * BLK:(hh + 1) * BLK, :])
        o_sc[...] = jnp.concatenate(o_cols, axis=1).astype(jnp.bfloat16)

    for pair in range(GRID // 2):
        @pl.when(jnp.logical_and(i >= GRID + 2 * pair,
                                 i < GRID + 2 * pair + 2))
        def _attn_pair(pair=pair):
            _attn((pair + 1) * 2 * BLK)

    @pl.when(i >= GRID)
    def _phase_b():
        j = i - GRID
        qrows = pl.ds(j * BLK, BLK)

        attn = jax.lax.dot_general(
            o_sc[...], wo_bf[...], _NT,
            preferred_element_type=jnp.float32) + bo_ref[...]
        r2 = attn + r1s[qrows, :]
        r2_out[...] = r2

        ms = jnp.mean(r2 * r2, axis=1, keepdims=True)
        h2 = (r2 * jax.lax.rsqrt(ms + EPS) * ln2_ref[...]).astype(
            jnp.bfloat16)

        logits = jax.lax.dot_general(
            h2, wr_bf[...], _NT,
            preferred_element_type=jnp.float32) + br_ref[...]
        m1 = jnp.max(logits, axis=1, keepdims=True)
        s2 = jnp.max(jnp.where(logits >= m1, _NEG, logits),
                     axis=1, keepdims=True)
        e2 = jnp.exp(s2 - m1)
        denom = 1.0 + e2
        factor = 1.0 / denom + e2 / denom

        gu = jax.lax.dot_general(
            h2, wgu_bf[...], _NT,
            preferred_element_type=jnp.float32) + bgu_ref[...]
        gate = gu[:, :I]
        up = gu[:, I:]
        x = gate * (up * jax.nn.sigmoid(up))
        eo = jax.lax.dot_general(
            x.astype(jnp.bfloat16), wd_bf[...], _NT,
            preferred_element_type=jnp.float32) + bd_ref[...]
        out_ref[...] = factor * eo


def kernel(positions, hidden_states, residual, w_qkv, b_qkv, w_o, b_o,
           w_router, b_router, w_gate_up, b_gate_up, w_down, b_down,
           ln1_w, ln2_w):
    f32 = jnp.float32
    bf16 = jnp.bfloat16
    pos = positions.astype(f32).reshape(S, 1)

    full = lambda shape: pl.BlockSpec(shape, lambda i: (0, 0))
    blk_a = lambda cols: pl.BlockSpec(
        (BLK, cols), lambda i: (jnp.minimum(i, GRID - 1), 0))
    blk_b = lambda cols: pl.BlockSpec(
        (BLK, cols), lambda i: (jnp.maximum(i - GRID, 0), 0))

    out, r2 = pl.pallas_call(
        _body,
        grid=(2 * GRID,),
        in_specs=[
            blk_a(1),
            blk_a(H),
            blk_a(H),
            full((Q_SIZE + 2 * KV_SIZE, H)),
            full((1, Q_SIZE + 2 * KV_SIZE)),
            full((1, H)),
            full((H, Q_SIZE)),
            full((1, H)),
            full((1, H)),
            full((E, H)),
            full((1, E)),
            full((2 * I, H)),
            full((1, 2 * I)),
            full((H, I)),
            full((1, H)),
        ],
        out_specs=[blk_b(H), blk_b(H)],
        out_shape=[
            jax.ShapeDtypeStruct((S, H), f32),
            jax.ShapeDtypeStruct((S, H), f32),
        ],
        scratch_shapes=[
            pltpu.VMEM((S, Q_SIZE), bf16),
            pltpu.VMEM((S, KV_SIZE), bf16),
            pltpu.VMEM((S, KV_SIZE), bf16),
            pltpu.VMEM((S, H), f32),
            pltpu.VMEM((BLK, Q_SIZE), bf16),
            pltpu.VMEM((Q_SIZE + 2 * KV_SIZE, H), bf16),
            pltpu.VMEM((H, Q_SIZE), bf16),
            pltpu.VMEM((E, H), bf16),
            pltpu.VMEM((2 * I, H), bf16),
            pltpu.VMEM((H, I), bf16),
        ],
    )(pos, hidden_states, residual, w_qkv,
      b_qkv.reshape(1, -1).astype(f32), ln1_w.reshape(1, H).astype(f32),
      w_o, b_o.reshape(1, H).astype(f32), ln2_w.reshape(1, H).astype(f32),
      w_router, b_router.reshape(1, E).astype(f32),
      w_gate_up, b_gate_up.reshape(1, 2 * I).astype(f32),
      w_down, b_down.reshape(1, H).astype(f32))

    return (out, r2)
